# Initial kernel scaffold; baseline (speedup 1.0000x reference)
import jax
import jax.numpy as jnp
from jax.experimental import pallas as pl


def kernel(edge_index, W1, b1, W2, b2, Wc, bc):
    raise NotImplementedError("write your pallas kernel here")


# speedup vs baseline: 20.3685x; 20.3685x over previous
import functools

import jax
import jax.numpy as jnp
from jax import lax
from jax.experimental import pallas as pl
from jax.experimental.pallas import tpu as pltpu
from jax.experimental.pallas import tpu_sc as plsc

N = 100000
E = 1600000
H = 32
NC, NS = 2, 16
NPAD = 100352
SL = NPAD // NS
EPT = E // NS
EPW = E // (NC * NS)
CD = 10000
CT = 10000
CG = 1000

_f32 = jnp.float32

_sc_mesh = plsc.VectorSubcoreMesh(
    core_axis_name="c", subcore_axis_name="s", num_cores=NC, num_subcores=NS)
_notc = pltpu.CompilerParams(use_tc_tiling_on_sc=False)


def _deg_body(src_hbm, dst_hbm, ones_hbm, zn_hbm, dout_hbm, din_hbm,
              idx_v, ones_v, sh):
    cid = lax.axis_index("c")
    sid = lax.axis_index("s")
    pltpu.sync_copy(zn_hbm.at[pl.ds(sid * SL, SL)], sh.at[pl.ds(sid * SL, SL)])
    pltpu.sync_copy(ones_hbm, ones_v)
    plsc.subcore_barrier()

    def run(idx_hbm):
        for i in range(EPT // CD):
            base = sid * EPT + i * CD
            pltpu.sync_copy(idx_hbm.at[pl.ds(base, CD)], idx_v)
            pltpu.sync_copy(ones_v, sh.at[idx_v], add=True)

    @pl.when(cid == 0)
    def _():
        run(src_hbm)

    @pl.when(cid == 1)
    def _():
        run(dst_hbm)

    plsc.subcore_barrier()

    @pl.when(cid == 0)
    def _():
        pltpu.sync_copy(sh.at[pl.ds(sid * SL, SL)],
                        dout_hbm.at[pl.ds(sid * SL, SL)])

    @pl.when(cid == 1)
    def _():
        pltpu.sync_copy(sh.at[pl.ds(sid * SL, SL)],
                        din_hbm.at[pl.ds(sid * SL, SL)])


_deg_call = pl.kernel(
    _deg_body,
    out_type=(jax.ShapeDtypeStruct((NPAD,), _f32),
              jax.ShapeDtypeStruct((NPAD,), _f32)),
    mesh=_sc_mesh,
    compiler_params=_notc,
    scratch_types=[
        pltpu.VMEM((CD,), jnp.int32),
        pltpu.VMEM((CD,), _f32),
        pltpu.VMEM_SHARED((NPAD,), _f32),
    ],
)


def _t_body(src_hbm, dst_hbm, s_hbm, zn_hbm, t0_hbm, t1_hbm,
            isv, idv, val_v, sh, sem):
    cid = lax.axis_index("c")
    sid = lax.axis_index("s")
    wid = cid * NS + sid
    pltpu.sync_copy(zn_hbm.at[pl.ds(sid * SL, SL)], sh.at[pl.ds(sid * SL, SL)])
    plsc.subcore_barrier()
    for i in range(EPW // CT):
        base = wid * EPW + i * CT
        pltpu.sync_copy(src_hbm.at[pl.ds(base, CT)], isv)
        pltpu.sync_copy(dst_hbm.at[pl.ds(base, CT)], idv)
        pltpu.async_copy(s_hbm.at[isv], val_v, sem).wait()
        pltpu.sync_copy(val_v, sh.at[idv], add=True)
    plsc.subcore_barrier()

    @pl.when(cid == 0)
    def _():
        pltpu.sync_copy(sh.at[pl.ds(sid * SL, SL)],
                        t0_hbm.at[pl.ds(sid * SL, SL)])

    @pl.when(cid == 1)
    def _():
        pltpu.sync_copy(sh.at[pl.ds(sid * SL, SL)],
                        t1_hbm.at[pl.ds(sid * SL, SL)])


_t_call = pl.kernel(
    _t_body,
    out_type=(jax.ShapeDtypeStruct((NPAD,), _f32),
              jax.ShapeDtypeStruct((NPAD,), _f32)),
    mesh=_sc_mesh,
    compiler_params=_notc,
    scratch_types=[
        pltpu.VMEM((CT,), jnp.int32),
        pltpu.VMEM((CT,), jnp.int32),
        pltpu.VMEM((CT,), _f32),
        pltpu.VMEM_SHARED((NPAD,), _f32),
        pltpu.SemaphoreType.DMA,
    ],
)


def _g_body(src_hbm, dst_hbm, y0_hbm, y1_hbm, zg_hbm, g0_hbm, g1_hbm,
            isv, idv, rows_v, sh, sem):
    cid = lax.axis_index("c")
    sid = lax.axis_index("s")
    pltpu.sync_copy(zg_hbm.at[pl.ds(sid * SL, SL)], sh.at[pl.ds(sid * SL, SL)])
    plsc.subcore_barrier()

    def run(y_hbm):
        def chunk(i, carry):
            base = sid * EPT + i * CG
            pltpu.sync_copy(src_hbm.at[pl.ds(base, CG)], isv)
            pltpu.sync_copy(dst_hbm.at[pl.ds(base, CG)], idv)
            pltpu.async_copy(y_hbm.at[isv], rows_v, sem).wait()
            pltpu.sync_copy(rows_v, sh.at[idv], add=True)
            return carry

        lax.fori_loop(0, EPT // CG, chunk, 0)

    @pl.when(cid == 0)
    def _():
        run(y0_hbm)

    @pl.when(cid == 1)
    def _():
        run(y1_hbm)

    plsc.subcore_barrier()

    @pl.when(cid == 0)
    def _():
        pltpu.sync_copy(sh.at[pl.ds(sid * SL, SL)],
                        g0_hbm.at[pl.ds(sid * SL, SL)])

    @pl.when(cid == 1)
    def _():
        pltpu.sync_copy(sh.at[pl.ds(sid * SL, SL)],
                        g1_hbm.at[pl.ds(sid * SL, SL)])


_g_call = pl.kernel(
    _g_body,
    out_type=(jax.ShapeDtypeStruct((NPAD, 16), _f32),
              jax.ShapeDtypeStruct((NPAD, 16), _f32)),
    mesh=_sc_mesh,
    compiler_params=_notc,
    scratch_types=[
        pltpu.VMEM((CG,), jnp.int32),
        pltpu.VMEM((CG,), jnp.int32),
        pltpu.VMEM((CG, 16), _f32),
        pltpu.VMEM_SHARED((NPAD, 16), _f32),
        pltpu.SemaphoreType.DMA,
    ],
)


def _pre_body(din_ref, dout_ref, s_ref, ns_ref, nd_ref):
    din = din_ref[...]
    dout = dout_ref[...]
    ns = lax.rsqrt(dout + 1.0)
    nd_ref[...] = lax.rsqrt(din + 1.0)
    ns_ref[...] = ns
    s_ref[...] = din * ns


_pre_call = pl.pallas_call(
    _pre_body,
    out_shape=(jax.ShapeDtypeStruct((NPAD // 128, 128), _f32),
               jax.ShapeDtypeStruct((NPAD // 128, 128), _f32),
               jax.ShapeDtypeStruct((NPAD // 128, 128), _f32)),
)


_BY = NPAD // 16


def _y_body(t0_ref, t1_ref, s_ref, ns_ref, nd_ref, w1_ref, b1_ref,
            y0_ref, y1_ref):
    u = (t0_ref[...] + t1_ref[...] + s_ref[...]) * nd_ref[...]
    pre = u * w1_ref[...] + b1_ref[...]
    y = ns_ref[...] * jnp.maximum(pre, 0.0)
    y0_ref[...] = y[:, :16]
    y1_ref[...] = y[:, 16:]


_y_call = pl.pallas_call(
    _y_body,
    grid=(16,),
    in_specs=[
        pl.BlockSpec((_BY, 1), lambda i: (i, 0)),
        pl.BlockSpec((_BY, 1), lambda i: (i, 0)),
        pl.BlockSpec((_BY, 1), lambda i: (i, 0)),
        pl.BlockSpec((_BY, 1), lambda i: (i, 0)),
        pl.BlockSpec((_BY, 1), lambda i: (i, 0)),
        pl.BlockSpec((1, H), lambda i: (0, 0)),
        pl.BlockSpec((1, H), lambda i: (0, 0)),
    ],
    out_specs=(pl.BlockSpec((_BY, 16), lambda i: (i, 0)),
               pl.BlockSpec((_BY, 16), lambda i: (i, 0))),
    out_shape=(jax.ShapeDtypeStruct((NPAD, 16), _f32),
               jax.ShapeDtypeStruct((NPAD, 16), _f32)),
)


_BO = 5000
_NBO = N // _BO


def _out_body(g0_ref, g1_ref, y0_ref, y1_ref, nd_ref, w2a_ref, w2b_ref,
              b2_ref, wc_ref, bc_ref, out_ref, acc_ref):
    i = pl.program_id(0)

    @pl.when(i == 0)
    def _():
        acc_ref[...] = jnp.zeros((1, H), _f32)

    z = (jnp.dot(g0_ref[...] + y0_ref[...], w2a_ref[...],
                 preferred_element_type=_f32)
         + jnp.dot(g1_ref[...] + y1_ref[...], w2b_ref[...],
                   preferred_element_type=_f32))
    h2 = jnp.maximum(z * nd_ref[...] + b2_ref[...], 0.0)
    acc_ref[...] += jnp.sum(h2, axis=0, keepdims=True)

    @pl.when(i == _NBO - 1)
    def _():
        hg = acc_ref[...] * (1.0 / N)
        out_ref[...] = jnp.dot(hg, wc_ref[...],
                               preferred_element_type=_f32) + bc_ref[...]


_out_call = pl.pallas_call(
    _out_body,
    grid=(_NBO,),
    in_specs=[
        pl.BlockSpec((_BO, 16), lambda i: (i, 0)),
        pl.BlockSpec((_BO, 16), lambda i: (i, 0)),
        pl.BlockSpec((_BO, 16), lambda i: (i, 0)),
        pl.BlockSpec((_BO, 16), lambda i: (i, 0)),
        pl.BlockSpec((_BO, 1), lambda i: (i, 0)),
        pl.BlockSpec((16, H), lambda i: (0, 0)),
        pl.BlockSpec((16, H), lambda i: (0, 0)),
        pl.BlockSpec((1, H), lambda i: (0, 0)),
        pl.BlockSpec((H, 10), lambda i: (0, 0)),
        pl.BlockSpec((1, 10), lambda i: (0, 0)),
    ],
    out_specs=pl.BlockSpec((1, 10), lambda i: (0, 0)),
    out_shape=jax.ShapeDtypeStruct((1, 10), _f32),
    scratch_shapes=[pltpu.VMEM((1, H), _f32)],
)


def kernel(edge_index, W1, b1, W2, b2, Wc, bc):
    src = edge_index[0]
    dst = edge_index[1]
    ones_c = jnp.ones((CD,), _f32)
    zn = jnp.zeros((NPAD,), _f32)
    zg = jnp.zeros((NPAD, 16), _f32)

    dout, din = _deg_call(src, dst, ones_c, zn)

    s2, ns2, nd2 = _pre_call(din.reshape(NPAD // 128, 128),
                             dout.reshape(NPAD // 128, 128))
    s = s2.reshape(NPAD)

    t0, t1 = _t_call(src, dst, s, zn)

    y0, y1 = _y_call(t0.reshape(NPAD, 1), t1.reshape(NPAD, 1),
                     s.reshape(NPAD, 1), ns2.reshape(NPAD, 1),
                     nd2.reshape(NPAD, 1), W1, b1.reshape(1, H))

    g0, g1 = _g_call(src, dst, y0, y1, zg)

    return _out_call(g0, g1, y0, y1, nd2.reshape(NPAD, 1),
                     W2[:16], W2[16:], b2.reshape(1, H), Wc,
                     bc.reshape(1, 10))


# speedup vs baseline: 34.0111x; 1.6698x over previous
import functools

import jax
import jax.numpy as jnp
from jax import lax
from jax.experimental import pallas as pl
from jax.experimental.pallas import tpu as pltpu
from jax.experimental.pallas import tpu_sc as plsc

N = 100000
E = 1600000
H = 32
NC, NS = 2, 16
NPAD = 100352
SL = NPAD // NS
EPT = E // NS
CG = 4000

_f32 = jnp.float32
_bf16 = jnp.bfloat16

_sc_mesh = plsc.VectorSubcoreMesh(
    core_axis_name="c", subcore_axis_name="s", num_cores=NC, num_subcores=NS)
_notc = pltpu.CompilerParams(use_tc_tiling_on_sc=False)


CA = 5000
_NCA = EPT // CA
_SUB = SL // 4


def _rsqrt_vec(x):
    xi = plsc.bitcast(x, jnp.int32)
    y = plsc.bitcast(0x5F3759DF - lax.shift_right_logical(xi, 1), _f32)
    for _ in range(3):
        y = y * (1.5 - 0.5 * x * y * y)
    return y


def _a_body(src_hbm, dst_hbm, wa_hbm, wb_hbm, ba_hbm, bb_hbm,
            ones_hbm, zn_hbm,
            y0_hbm, y1_hbm, nd_hbm,
            isv, idv, val_v, isv1, idv1, val_v1, ones_v,
            a_v, b_v, ns_v, nd_v, s_v,
            wa_v, wb_v, ba_v, bb_v, ybuf, ybuf1, sh_din, sh_dout, sh_t, sh_s,
            isem0, isem1, gsem0, gsem1, ssem0, ssem1):
    cid = lax.axis_index("c")
    sid = lax.axis_index("s")
    sl0 = sid * SL

    def start_idx(c, isb, idb, isem):
        base = sid * EPT + c * CA
        pltpu.async_copy(src_hbm.at[pl.ds(base, CA)], isb, isem)
        pltpu.async_copy(dst_hbm.at[pl.ds(base, CA)], idb, isem)

    def wait_idx(isb, idb, isem):
        pltpu.make_async_copy(src_hbm.at[pl.ds(0, CA)], isb, isem).wait()
        pltpu.make_async_copy(dst_hbm.at[pl.ds(0, CA)], idb, isem).wait()
    pltpu.sync_copy(zn_hbm.at[pl.ds(sl0, SL)], sh_din.at[pl.ds(sl0, SL)])
    pltpu.sync_copy(zn_hbm.at[pl.ds(sl0, SL)], sh_dout.at[pl.ds(sl0, SL)])
    pltpu.sync_copy(zn_hbm.at[pl.ds(sl0, SL)], sh_t.at[pl.ds(sl0, SL)])
    pltpu.sync_copy(ones_hbm, ones_v)
    pltpu.sync_copy(wa_hbm, wa_v)
    pltpu.sync_copy(wb_hbm, wb_v)
    pltpu.sync_copy(ba_hbm, ba_v)
    pltpu.sync_copy(bb_hbm, bb_v)
    plsc.subcore_barrier()

    def cnt_start(isb, idb, ssem):
        pltpu.async_copy(ones_v, sh_dout.at[isb], ssem, add=True)
        pltpu.async_copy(ones_v, sh_din.at[idb], ssem, add=True)

    def cnt_wait(isb, idb, ssem):
        pltpu.make_async_copy(ones_v, sh_dout.at[isb], ssem).wait()
        pltpu.make_async_copy(ones_v, sh_din.at[idb], ssem).wait()

    start_idx(0, isv, idv, isem0)

    def bpair(i, carry):
        wait_idx(isv, idv, isem0)
        cnt_start(isv, idv, ssem0)
        start_idx(2 * i + 1, isv1, idv1, isem1)
        cnt_wait(isv, idv, ssem0)

        @pl.when(2 * i + 2 < _NCA)
        def _():
            start_idx(2 * i + 2, isv, idv, isem0)

        wait_idx(isv1, idv1, isem1)
        cnt_start(isv1, idv1, ssem1)
        cnt_wait(isv1, idv1, ssem1)
        return carry

    lax.fori_loop(0, _NCA // 2, bpair, 0)
    plsc.subcore_barrier()

    pltpu.sync_copy(sh_din.at[pl.ds(sl0, SL)], a_v)
    pltpu.sync_copy(sh_dout.at[pl.ds(sl0, SL)], b_v)

    def cvec(i, carry):
        din = a_v[pl.ds(i * 16, 16)]
        dout = b_v[pl.ds(i * 16, 16)]
        ns = _rsqrt_vec(dout + 1.0)
        ns_v[pl.ds(i * 16, 16)] = ns
        nd_v[pl.ds(i * 16, 16)] = _rsqrt_vec(din + 1.0)
        s_v[pl.ds(i * 16, 16)] = din * ns
        return carry

    lax.fori_loop(0, SL // 16, cvec, 0)
    pltpu.sync_copy(s_v, sh_s.at[pl.ds(sl0, SL)])

    @pl.when(cid == 0)
    def _():
        pltpu.sync_copy(nd_v, nd_hbm.at[pl.ds(sl0, SL)])

    plsc.subcore_barrier()

    start_idx(0, isv, idv, isem0)

    def dpair(i, carry):
        wait_idx(isv, idv, isem0)
        pltpu.async_copy(sh_s.at[isv], val_v, gsem0)
        start_idx(2 * i + 1, isv1, idv1, isem1)
        pltpu.make_async_copy(sh_s.at[isv], val_v, gsem0).wait()
        pltpu.async_copy(val_v, sh_t.at[idv], ssem0, add=True)
        wait_idx(isv1, idv1, isem1)
        pltpu.async_copy(sh_s.at[isv1], val_v1, gsem1)
        pltpu.make_async_copy(val_v, sh_t.at[idv], ssem0).wait()

        @pl.when(2 * i + 2 < _NCA)
        def _():
            start_idx(2 * i + 2, isv, idv, isem0)

        pltpu.make_async_copy(sh_s.at[isv1], val_v1, gsem1).wait()
        pltpu.async_copy(val_v1, sh_t.at[idv1], ssem1, add=True)
        pltpu.make_async_copy(val_v1, sh_t.at[idv1], ssem1).wait()
        return carry

    lax.fori_loop(0, _NCA // 2, dpair, 0)
    plsc.subcore_barrier()

    pltpu.sync_copy(sh_t.at[pl.ds(sl0, SL)], a_v)
    hi_mask = lax.iota(jnp.int32, 16) >= 8

    def ephase(y_hbm, off):
        wa = wa_v[pl.ds(off, 16)]
        wb = wb_v[pl.ds(off, 16)]
        ba = ba_v[pl.ds(off, 16)]
        bb = bb_v[pl.ds(off, 16)]
        bufs = (ybuf, ybuf1)
        for sub in range(4):
            yb = bufs[sub % 2]

            def rowvec(i, carry):
                n0 = sub * _SUB + i * 16
                uvec = (a_v[pl.ds(n0, 16)] + s_v[pl.ds(n0, 16)]) \
                    * nd_v[pl.ds(n0, 16)]
                nsvec = ns_v[pl.ds(n0, 16)]
                for k in range(8):
                    un2 = jnp.where(hi_mask, uvec[2 * k + 1], uvec[2 * k])
                    ns2 = jnp.where(hi_mask, nsvec[2 * k + 1], nsvec[2 * k])
                    ra = ns2 * jnp.maximum(un2 * wa + ba, 0.0)
                    rb = ns2 * jnp.maximum(un2 * wb + bb, 0.0)
                    yb[pl.ds((i * 16 + 2 * k) * 16, 32)] = plsc.pack(
                        ra, rb, format=plsc.PackFormat.INTERLEAVED)
                return carry

            if sub >= 2:
                pltpu.make_async_copy(
                    yb,
                    y_hbm.at[pl.ds((sl0 + (sub - 2) * _SUB) * 16, _SUB * 16)],
                    gsem0).wait()
            lax.fori_loop(0, _SUB // 16, rowvec, 0)
            pltpu.async_copy(
                yb, y_hbm.at[pl.ds((sl0 + sub * _SUB) * 16, _SUB * 16)],
                gsem0)
        pltpu.make_async_copy(
            ybuf, y_hbm.at[pl.ds((sl0 + 2 * _SUB) * 16, _SUB * 16)],
            gsem0).wait()
        pltpu.make_async_copy(
            ybuf1, y_hbm.at[pl.ds((sl0 + 3 * _SUB) * 16, _SUB * 16)],
            gsem0).wait()

    @pl.when(cid == 0)
    def _():
        ephase(y0_hbm, 0)

    @pl.when(cid == 1)
    def _():
        ephase(y1_hbm, 16)


_a_call = pl.kernel(
    _a_body,
    out_type=(jax.ShapeDtypeStruct((NPAD * 16,), _bf16),
              jax.ShapeDtypeStruct((NPAD * 16,), _bf16),
              jax.ShapeDtypeStruct((NPAD,), _f32)),
    mesh=_sc_mesh,
    compiler_params=pltpu.CompilerParams(
        use_tc_tiling_on_sc=False, needs_layout_passes=False),
    scratch_types=[
        pltpu.VMEM((CA,), jnp.int32),
        pltpu.VMEM((CA,), jnp.int32),
        pltpu.VMEM((CA,), _f32),
        pltpu.VMEM((CA,), jnp.int32),
        pltpu.VMEM((CA,), jnp.int32),
        pltpu.VMEM((CA,), _f32),
        pltpu.VMEM((CA,), _f32),
        pltpu.VMEM((SL,), _f32),
        pltpu.VMEM((SL,), _f32),
        pltpu.VMEM((SL,), _f32),
        pltpu.VMEM((SL,), _f32),
        pltpu.VMEM((SL,), _f32),
        pltpu.VMEM((H,), _f32),
        pltpu.VMEM((H,), _f32),
        pltpu.VMEM((H,), _f32),
        pltpu.VMEM((H,), _f32),
        pltpu.VMEM((_SUB * 16,), _bf16),
        pltpu.VMEM((_SUB * 16,), _bf16),
        pltpu.VMEM_SHARED((NPAD,), _f32),
        pltpu.VMEM_SHARED((NPAD,), _f32),
        pltpu.VMEM_SHARED((NPAD,), _f32),
        pltpu.VMEM_SHARED((NPAD,), _f32),
        pltpu.SemaphoreType.DMA,
        pltpu.SemaphoreType.DMA,
        pltpu.SemaphoreType.DMA,
        pltpu.SemaphoreType.DMA,
        pltpu.SemaphoreType.DMA,
        pltpu.SemaphoreType.DMA,
    ],
)


_NCH = EPT // CG
_NPAIR = _NCH // 2


def _g_body(src_hbm, dst_hbm, y0_hbm, y1_hbm, zg_hbm, g0_hbm, g1_hbm,
            isv0, idv0, rows0, isv1, idv1, rows1, sh,
            isem0, gsem0, ssem0, isem1, gsem1, ssem1):
    cid = lax.axis_index("c")
    sid = lax.axis_index("s")
    pltpu.sync_copy(zg_hbm.at[pl.ds(sid * SL, SL)], sh.at[pl.ds(sid * SL, SL)])
    plsc.subcore_barrier()

    def run(y_hbm):
        def start_idx(c, isv, idv, isem):
            base = sid * EPT + c * CG
            pltpu.async_copy(src_hbm.at[pl.ds(base, CG)], isv, isem)
            pltpu.async_copy(dst_hbm.at[pl.ds(base, CG)], idv, isem)

        def wait_idx(isv, idv, isem):
            pltpu.make_async_copy(src_hbm.at[pl.ds(0, CG)], isv, isem).wait()
            pltpu.make_async_copy(dst_hbm.at[pl.ds(0, CG)], idv, isem).wait()

        start_idx(0, isv0, idv0, isem0)

        def pair(i, carry):
            c0 = 2 * i
            wait_idx(isv0, idv0, isem0)
            pltpu.async_copy(y_hbm.at[isv0], rows0, gsem0)
            start_idx(c0 + 1, isv1, idv1, isem1)
            pltpu.make_async_copy(y_hbm.at[isv0], rows0, gsem0).wait()
            pltpu.async_copy(rows0, sh.at[idv0], ssem0, add=True)
            wait_idx(isv1, idv1, isem1)
            pltpu.async_copy(y_hbm.at[isv1], rows1, gsem1)
            pltpu.make_async_copy(rows0, sh.at[idv0], ssem0).wait()

            @pl.when(c0 + 2 < _NCH)
            def _():
                start_idx(c0 + 2, isv0, idv0, isem0)

            pltpu.make_async_copy(y_hbm.at[isv1], rows1, gsem1).wait()
            pltpu.async_copy(rows1, sh.at[idv1], ssem1, add=True)
            pltpu.make_async_copy(rows1, sh.at[idv1], ssem1).wait()
            return carry

        lax.fori_loop(0, _NPAIR, pair, 0)

        if _NCH % 2 == 1:
            wait_idx(isv0, idv0, isem0)
            pltpu.async_copy(y_hbm.at[isv0], rows0, gsem0)
            pltpu.make_async_copy(y_hbm.at[isv0], rows0, gsem0).wait()
            pltpu.async_copy(rows0, sh.at[idv0], ssem0, add=True)
            pltpu.make_async_copy(rows0, sh.at[idv0], ssem0).wait()

    @pl.when(cid == 0)
    def _():
        run(y0_hbm)

    @pl.when(cid == 1)
    def _():
        run(y1_hbm)

    plsc.subcore_barrier()

    @pl.when(cid == 0)
    def _():
        pltpu.sync_copy(sh.at[pl.ds(sid * SL, SL)],
                        g0_hbm.at[pl.ds(sid * SL, SL)])

    @pl.when(cid == 1)
    def _():
        pltpu.sync_copy(sh.at[pl.ds(sid * SL, SL)],
                        g1_hbm.at[pl.ds(sid * SL, SL)])


_g_call = pl.kernel(
    _g_body,
    out_type=(jax.ShapeDtypeStruct((NPAD, 16), _bf16),
              jax.ShapeDtypeStruct((NPAD, 16), _bf16)),
    mesh=_sc_mesh,
    compiler_params=_notc,
    scratch_types=[
        pltpu.VMEM((CG,), jnp.int32),
        pltpu.VMEM((CG,), jnp.int32),
        pltpu.VMEM((CG, 16), _bf16),
        pltpu.VMEM((CG,), jnp.int32),
        pltpu.VMEM((CG,), jnp.int32),
        pltpu.VMEM((CG, 16), _bf16),
        pltpu.VMEM_SHARED((NPAD, 16), _bf16),
        pltpu.SemaphoreType.DMA,
        pltpu.SemaphoreType.DMA,
        pltpu.SemaphoreType.DMA,
        pltpu.SemaphoreType.DMA,
        pltpu.SemaphoreType.DMA,
        pltpu.SemaphoreType.DMA,
    ],
)


_BO = 5000
_NBO = N // _BO


def _out_body(g0_ref, g1_ref, y0_ref, y1_ref, nd_ref, w2a_ref, w2b_ref,
              b2_ref, wc_ref, bc_ref, out_ref, acc_ref):
    i = pl.program_id(0)

    @pl.when(i == 0)
    def _():
        acc_ref[...] = jnp.zeros((1, H), _f32)

    za = (g0_ref[...] + y0_ref[...].astype(_f32))
    zb = (g1_ref[...] + y1_ref[...].astype(_f32))
    z = (jnp.dot(za, w2a_ref[...], preferred_element_type=_f32)
         + jnp.dot(zb, w2b_ref[...], preferred_element_type=_f32))
    h2 = jnp.maximum(z * nd_ref[...] + b2_ref[...], 0.0)
    acc_ref[...] += jnp.sum(h2, axis=0, keepdims=True)

    @pl.when(i == _NBO - 1)
    def _():
        hg = acc_ref[...] * (1.0 / N)
        out_ref[...] = jnp.dot(hg, wc_ref[...],
                               preferred_element_type=_f32) + bc_ref[...]


_out_call = pl.pallas_call(
    _out_body,
    grid=(_NBO,),
    in_specs=[
        pl.BlockSpec((_BO, 16), lambda i: (i, 0)),
        pl.BlockSpec((_BO, 16), lambda i: (i, 0)),
        pl.BlockSpec((_BO, 16), lambda i: (i, 0)),
        pl.BlockSpec((_BO, 16), lambda i: (i, 0)),
        pl.BlockSpec((_BO, 1), lambda i: (i, 0)),
        pl.BlockSpec((16, H), lambda i: (0, 0)),
        pl.BlockSpec((16, H), lambda i: (0, 0)),
        pl.BlockSpec((1, H), lambda i: (0, 0)),
        pl.BlockSpec((H, 10), lambda i: (0, 0)),
        pl.BlockSpec((1, 10), lambda i: (0, 0)),
    ],
    out_specs=pl.BlockSpec((1, 10), lambda i: (0, 0)),
    out_shape=jax.ShapeDtypeStruct((1, 10), _f32),
    scratch_shapes=[pltpu.VMEM((1, H), _f32)],
)


def kernel(edge_index, W1, b1, W2, b2, Wc, bc):
    src = edge_index[0]
    dst = edge_index[1]
    ones_c = jnp.ones((CA,), _f32)
    zn = jnp.zeros((NPAD,), _f32)
    zg = jnp.zeros((NPAD, 16), _bf16)

    w1 = W1.reshape(H)
    wa = jnp.concatenate([jnp.tile(w1[0:8], 2), jnp.tile(w1[16:24], 2)])
    wb = jnp.concatenate([jnp.tile(w1[8:16], 2), jnp.tile(w1[24:32], 2)])
    ba = jnp.concatenate([jnp.tile(b1[0:8], 2), jnp.tile(b1[16:24], 2)])
    bb = jnp.concatenate([jnp.tile(b1[8:16], 2), jnp.tile(b1[24:32], 2)])

    y0f, y1f, nd = _a_call(src, dst, wa, wb, ba, bb, ones_c, zn)
    y0 = y0f.reshape(NPAD, 16)
    y1 = y1f.reshape(NPAD, 16)

    g0, g1 = _g_call(src, dst, y0, y1, zg)

    perm = jnp.stack([jnp.arange(8), jnp.arange(8) + 8], axis=1).reshape(16)
    w2a = W2[perm]
    w2b = W2[perm + 16]

    return _out_call(g0, g1, y0, y1, nd.reshape(NPAD, 1),
                     w2a, w2b, b2.reshape(1, H), Wc,
                     bc.reshape(1, 10))
